# baseline (speedup 1.0000x reference)
import functools
import jax
import jax.numpy as jnp
from jax import lax
from jax.experimental import pallas as pl
from jax.experimental.pallas import tpu as pltpu

B, N, C, K = 4, 2048, 128, 16
NB1 = 512
MB = 128
NSTEPS1 = N // NB1
M_ROWS = B * N * K


def _k1_body(xyz_blk, xyz_full, xyzT_full, feat_blk, Wq, Wk, Wv, attn_w1,
             aq_ref, tab_ref, idx_ref, rel_ref, stats_ref):
    i = pl.program_id(0)

    WqA = jnp.dot(Wq[...], attn_w1[...], preferred_element_type=jnp.float32)
    WkA = jnp.dot(Wk[...], attn_w1[...], preferred_element_type=jnp.float32)
    f = feat_blk[...]
    aq_ref[...] = jnp.dot(f, WqA, preferred_element_type=jnp.float32)
    tab_ref[:, 0:C] = jnp.dot(f, WkA, preferred_element_type=jnp.float32)
    tab_ref[:, C:2 * C] = jnp.dot(f, Wv[...], preferred_element_type=jnp.float32)

    xb = xyz_blk[...][0]
    xf = xyz_full[...][0]
    xfT = xyzT_full[...][0]
    sqb = jnp.sum(xb * xb, axis=1, keepdims=True)
    sqf = jnp.sum(xfT * xfT, axis=0, keepdims=True)
    cross = jnp.dot(xb, xfT, preferred_element_type=jnp.float32)
    d = sqb + sqf - 2.0 * cross

    lanes = lax.broadcasted_iota(jnp.int32, (NB1, N), 1)

    selfcol = lax.broadcasted_iota(jnp.int32, (NB1, 1), 0) + i * NB1
    d = jnp.where(lanes == selfcol, jnp.inf, d)
    idx_cols = [selfcol]
    rel_cols = [jnp.zeros((NB1, 3), jnp.float32)]
    s1 = None
    for _ in range(K - 1):
        m = jnp.min(d, axis=1, keepdims=True)
        sel = jnp.min(jnp.where(d == m, lanes, N), axis=1, keepdims=True)
        onehot = lanes == sel
        d = jnp.where(onehot, jnp.inf, d)
        nxyz = jnp.dot(onehot.astype(jnp.float32), xf,
                       preferred_element_type=jnp.float32)
        rel_j = nxyz - xb
        idx_cols.append(sel)
        rel_cols.append(rel_j)
        p1 = jnp.sum(rel_j, axis=0, keepdims=True)
        rx = rel_j[:, 0:1]
        ry = rel_j[:, 1:2]
        rz = rel_j[:, 2:3]
        p2 = jnp.concatenate([
            jnp.sum(rx * rx, axis=0, keepdims=True),
            jnp.sum(rx * ry, axis=0, keepdims=True),
            jnp.sum(rx * rz, axis=0, keepdims=True),
            jnp.sum(ry * ry, axis=0, keepdims=True),
            jnp.sum(ry * rz, axis=0, keepdims=True),
            jnp.sum(rz * rz, axis=0, keepdims=True),
        ], axis=1)
        pj = jnp.concatenate([p1, p2], axis=1)
        s1 = pj if s1 is None else s1 + pj

    idx_ref[...] = jnp.concatenate(idx_cols, axis=1)
    rel_ref[...] = jnp.concatenate(rel_cols, axis=1)

    svec = jnp.pad(s1, ((0, 7), (0, 128 - 9)))
    first = i == 0

    @pl.when(first)
    def _init():
        stats_ref[...] = svec

    @pl.when(jnp.logical_not(first))
    def _acc():
        stats_ref[...] = stats_ref[...] + svec


def _k2_body(g_ref, rel_ref, aq_ref, s0_ref, s1_ref, s2_ref, s3_ref,
             pos_w1, pos_b1, pos_g, pos_beta, pos_w2, pos_b2,
             attn_w1, attn_b1,
             z_ref, vpos_ref, zstats_ref):
    step = pl.program_id(0)
    Minv = 1.0 / float(M_ROWS)

    def st(j):
        return s0_ref[0, j] + s1_ref[0, j] + s2_ref[0, j] + s3_ref[0, j]

    mux = st(0) * Minv
    muy = st(1) * Minv
    muz = st(2) * Minv
    cxx = st(3) * Minv - mux * mux
    cxy = st(4) * Minv - mux * muy
    cxz = st(5) * Minv - mux * muz
    cyy = st(6) * Minv - muy * muy
    cyz = st(7) * Minv - muy * muz
    czz = st(8) * Minv - muz * muz

    w1 = pos_w1[...]
    w1x = w1[0:1, :]
    w1y = w1[1:2, :]
    w1z = w1[2:3, :]
    var_h = (cxx * w1x * w1x + cyy * w1y * w1y + czz * w1z * w1z
             + 2.0 * (cxy * w1x * w1y + cxz * w1x * w1z + cyz * w1y * w1z))
    mean_h = mux * w1x + muy * w1y + muz * w1z + pos_b1[...]
    inv = lax.rsqrt(var_h + 1e-5)
    scale = pos_g[...] * inv
    shift = pos_beta[...] - mean_h * scale

    rel = rel_ref[...]
    rx = rel[:, 0:1]
    ry = rel[:, 1:2]
    rz = rel[:, 2:3]
    h = rx * w1x + ry * w1y + rz * w1z + pos_b1[...]
    h_act = jnp.maximum(h * scale + shift, 0.0)

    Wpz = jnp.dot(pos_w2[...], attn_w1[...], preferred_element_type=jnp.float32)
    pos = jnp.dot(h_act, pos_w2[...], preferred_element_type=jnp.float32)
    posz = jnp.dot(h_act, Wpz, preferred_element_type=jnp.float32)

    g = g_ref[...]
    ak = g[:, 0:C]
    vv = g[:, C:2 * C]

    aq = aq_ref[...]
    aq_rows = jnp.broadcast_to(aq.reshape(MB, 1, C), (MB, K, C)).reshape(MB * K, C)

    bz = attn_b1[...] + jnp.dot(pos_b2[...], attn_w1[...],
                                preferred_element_type=jnp.float32)
    z = aq_rows - ak + posz + bz
    z_ref[...] = z
    vpos_ref[...] = vv + pos + pos_b2[...]

    zsum = jnp.sum(z, axis=0, keepdims=True)
    zsq = jnp.sum(z * z, axis=0, keepdims=True)

    @pl.when(step == 0)
    def _init():
        zstats_ref[...] = jnp.zeros_like(zstats_ref)

    zstats_ref[0:1, :] += zsum
    zstats_ref[1:2, :] += zsq


def _k3_body(z_ref, vpos_ref, z0_ref, z1_ref, z2_ref, z3_ref,
             attn_g, attn_beta, attn_w2, attn_b2, Wout, out_ref):
    Minv = 1.0 / float(M_ROWS)
    zs = z0_ref[...] + z1_ref[...] + z2_ref[...] + z3_ref[...]
    mean_z = zs[0:1, :] * Minv
    var_z = zs[1:2, :] * Minv - mean_z * mean_z
    inv = lax.rsqrt(var_z + 1e-5)
    scale = attn_g[...] * inv
    shift = attn_beta[...] - mean_z * scale

    z_act = jnp.maximum(z_ref[...] * scale + shift, 0.0)
    a = jnp.dot(z_act, attn_w2[...], preferred_element_type=jnp.float32) \
        + attn_b2[...]
    a3 = a.reshape(MB, K, C)
    mx = jnp.max(a3, axis=1, keepdims=True)
    e = jnp.exp(a3 - mx)
    s = jnp.sum(e, axis=1, keepdims=True)
    attn = e / s
    vp = vpos_ref[...].reshape(MB, K, C)
    o = jnp.sum(attn * vp, axis=1)
    out_ref[...] = jnp.dot(o, Wout[...], preferred_element_type=jnp.float32)


def _sc_gather(table, idxflat, nrows):
    from jax.experimental.pallas import tpu_sc as plsc
    info = plsc.get_sparse_core_info()
    NC, NS = info.num_cores, info.num_subcores
    NW = NC * NS
    rows_per_w = nrows // NW
    CHUNK = 128
    n_chunks = rows_per_w // CHUNK
    n_pairs = n_chunks // 2
    mesh = plsc.VectorSubcoreMesh(core_axis_name="c", subcore_axis_name="s")

    @functools.partial(
        pl.kernel, mesh=mesh,
        out_type=jax.ShapeDtypeStruct((nrows, 2 * C), jnp.float32),
        scratch_types=[
            pltpu.VMEM((CHUNK,), jnp.int32),
            pltpu.VMEM((2, CHUNK, 2 * C), jnp.float32),
            pltpu.SemaphoreType.DMA,
            pltpu.SemaphoreType.DMA,
            pltpu.SemaphoreType.DMA,
        ],
    )
    def gk(table_hbm, idx_hbm, out_hbm, idx_v, rows_v, sem_g, sem_w0, sem_w1):
        wid = lax.axis_index("s") * NC + lax.axis_index("c")
        base = wid * rows_per_w

        def body(t, carry):
            off0 = pl.multiple_of(base + (2 * t) * CHUNK, CHUNK)
            off1 = pl.multiple_of(base + (2 * t + 1) * CHUNK, CHUNK)

            @pl.when(t >= 1)
            def _drain():
                pltpu.make_async_copy(
                    rows_v.at[0], out_hbm.at[pl.ds(off0, CHUNK)], sem_w0).wait()
                pltpu.make_async_copy(
                    rows_v.at[1], out_hbm.at[pl.ds(off1, CHUNK)], sem_w1).wait()

            pltpu.sync_copy(idx_hbm.at[pl.ds(off0, CHUNK)], idx_v)
            pltpu.async_copy(table_hbm.at[idx_v], rows_v.at[0], sem_g).wait()
            pltpu.async_copy(rows_v.at[0], out_hbm.at[pl.ds(off0, CHUNK)], sem_w0)

            pltpu.sync_copy(idx_hbm.at[pl.ds(off1, CHUNK)], idx_v)
            pltpu.async_copy(table_hbm.at[idx_v], rows_v.at[1], sem_g).wait()
            pltpu.async_copy(rows_v.at[1], out_hbm.at[pl.ds(off1, CHUNK)], sem_w1)
            return carry

        lax.fori_loop(0, n_pairs, body, 0)
        pltpu.make_async_copy(
            rows_v.at[0], out_hbm.at[pl.ds(base, CHUNK)], sem_w0).wait()
        pltpu.make_async_copy(
            rows_v.at[1], out_hbm.at[pl.ds(base, CHUNK)], sem_w1).wait()

    return gk(table, idxflat)


_WCC = pl.BlockSpec((C, C), lambda i: (0, 0))
_W1C = pl.BlockSpec((1, C), lambda i: (0, 0))
_WST = pl.BlockSpec((8, 128), lambda i: (0, 0))
NK = N * K


def kernel(xyz, feat, Wq, Wk, Wv, Wout, pos_w1, pos_b1, pos_g, pos_beta,
           pos_w2, pos_b2, attn_w1, attn_b1, attn_g, attn_beta,
           attn_w2, attn_b2):
    xyzT = xyz.transpose(0, 2, 1)
    pos_b1r = pos_b1.reshape(1, C)
    pos_gr = pos_g.reshape(1, C)
    pos_betar = pos_beta.reshape(1, C)
    pos_b2r = pos_b2.reshape(1, C)
    attn_b1r = attn_b1.reshape(1, C)
    attn_gr = attn_g.reshape(1, C)
    attn_betar = attn_beta.reshape(1, C)
    attn_b2r = attn_b2.reshape(1, C)

    k1 = pl.pallas_call(
        _k1_body,
        grid=(NSTEPS1,),
        in_specs=[
            pl.BlockSpec((1, NB1, 3), lambda i: (0, i, 0)),
            pl.BlockSpec((1, N, 3), lambda i: (0, 0, 0)),
            pl.BlockSpec((1, 3, N), lambda i: (0, 0, 0)),
            pl.BlockSpec((NB1, C), lambda i: (i, 0)),
            pl.BlockSpec((C, C), lambda i: (0, 0)),
            pl.BlockSpec((C, C), lambda i: (0, 0)),
            pl.BlockSpec((C, C), lambda i: (0, 0)),
            pl.BlockSpec((C, C), lambda i: (0, 0)),
        ],
        out_specs=[
            pl.BlockSpec((NB1, C), lambda i: (i, 0)),
            pl.BlockSpec((NB1, 2 * C), lambda i: (i, 0)),
            pl.BlockSpec((NB1, K), lambda i: (i, 0)),
            pl.BlockSpec((NB1, 3 * K), lambda i: (i, 0)),
            pl.BlockSpec((8, 128), lambda i: (0, 0)),
        ],
        out_shape=[
            jax.ShapeDtypeStruct((N, C), jnp.float32),
            jax.ShapeDtypeStruct((N, 2 * C), jnp.float32),
            jax.ShapeDtypeStruct((N, K), jnp.int32),
            jax.ShapeDtypeStruct((N, 3 * K), jnp.float32),
            jax.ShapeDtypeStruct((8, 128), jnp.float32),
        ],
    )

    aqs, tabs, idxs, rels, stats = [], [], [], [], []
    for b in range(B):
        aq, tab, idx, rel48, st = k1(xyz[b:b + 1], xyz[b:b + 1],
                                     xyzT[b:b + 1], feat[b],
                                     Wq, Wk, Wv, attn_w1)
        aqs.append(aq)
        tabs.append(tab)
        idxs.append(idx)
        rels.append(rel48)
        stats.append(st)

    gs = [_sc_gather(tabs[b], idxs[b].reshape(NK), NK) for b in range(B)]

    nsteps2 = N // MB
    k2 = pl.pallas_call(
        _k2_body,
        grid=(nsteps2,),
        in_specs=[
            pl.BlockSpec((MB * K, 2 * C), lambda i: (i, 0)),
            pl.BlockSpec((MB * K, 3), lambda i: (i, 0)),
            pl.BlockSpec((MB, C), lambda i: (i, 0)),
            _WST, _WST, _WST, _WST,
            pl.BlockSpec((3, C), lambda i: (0, 0)),
            _W1C, _W1C, _W1C, _WCC, _W1C, _WCC, _W1C,
        ],
        out_specs=[
            pl.BlockSpec((MB * K, C), lambda i: (i, 0)),
            pl.BlockSpec((MB * K, C), lambda i: (i, 0)),
            pl.BlockSpec((8, 128), lambda i: (0, 0)),
        ],
        out_shape=[
            jax.ShapeDtypeStruct((NK, C), jnp.float32),
            jax.ShapeDtypeStruct((NK, C), jnp.float32),
            jax.ShapeDtypeStruct((8, 128), jnp.float32),
        ],
    )

    zs, vps, zsts = [], [], []
    for b in range(B):
        z, vpos, zst = k2(gs[b], rels[b].reshape(NK, 3), aqs[b],
                          stats[0], stats[1], stats[2], stats[3],
                          pos_w1, pos_b1r, pos_gr, pos_betar,
                          pos_w2, pos_b2r, attn_w1, attn_b1r)
        zs.append(z)
        vps.append(vpos)
        zsts.append(zst)

    k3 = pl.pallas_call(
        _k3_body,
        grid=(nsteps2,),
        in_specs=[
            pl.BlockSpec((MB * K, C), lambda i: (i, 0)),
            pl.BlockSpec((MB * K, C), lambda i: (i, 0)),
            _WST, _WST, _WST, _WST,
            _W1C, _W1C, _WCC, _W1C, _WCC,
        ],
        out_specs=pl.BlockSpec((MB, C), lambda i: (i, 0)),
        out_shape=jax.ShapeDtypeStruct((N, C), jnp.float32),
    )

    outs = [k3(zs[b], vps[b], zsts[0], zsts[1], zsts[2], zsts[3],
               attn_gr, attn_betar, attn_w2, attn_b2r, Wout)
            for b in range(B)]
    return jnp.stack(outs, axis=0)


# speedup vs baseline: 1.0669x; 1.0669x over previous
import functools
import jax
import jax.numpy as jnp
from jax import lax
from jax.experimental import pallas as pl
from jax.experimental.pallas import tpu as pltpu

B, N, C, K = 4, 2048, 128, 16
NB1 = 256
MB = 128
NSTEPS1 = N // NB1
M_ROWS = B * N * K


def _k1_body(xyz_blk, xyz_full, xyzT_full, feat_blk, Wq, Wk, Wv, attn_w1,
             aq_ref, tab_ref, idx_ref, rel_ref, stats_ref):
    i = pl.program_id(0)

    WqA = jnp.dot(Wq[...], attn_w1[...], preferred_element_type=jnp.float32)
    WkA = jnp.dot(Wk[...], attn_w1[...], preferred_element_type=jnp.float32)
    f = feat_blk[...]
    aq_ref[...] = jnp.dot(f, WqA, preferred_element_type=jnp.float32)
    tab_ref[:, 0:C] = jnp.dot(f, WkA, preferred_element_type=jnp.float32)
    tab_ref[:, C:2 * C] = jnp.dot(f, Wv[...], preferred_element_type=jnp.float32)

    xb = xyz_blk[...][0]
    xf = xyz_full[...][0]
    xfT = xyzT_full[...][0]
    sqb = jnp.sum(xb * xb, axis=1, keepdims=True)
    sqf = jnp.sum(xfT * xfT, axis=0, keepdims=True)
    cross = jnp.dot(xb, xfT, preferred_element_type=jnp.float32)
    d = sqb + sqf - 2.0 * cross

    lanes = lax.broadcasted_iota(jnp.int32, (NB1, N), 1)

    selfcol = lax.broadcasted_iota(jnp.int32, (NB1, 1), 0) + i * NB1
    d = jnp.where(lanes == selfcol, jnp.inf, d)
    idx_cols = [selfcol]
    rel_cols = [jnp.zeros((NB1, 3), jnp.float32)]
    s1 = None
    for _ in range(K - 1):
        m = jnp.min(d, axis=1, keepdims=True)
        sel = jnp.min(jnp.where(d == m, lanes, N), axis=1, keepdims=True)
        onehot = lanes == sel
        d = jnp.where(onehot, jnp.inf, d)
        nxyz = jnp.dot(onehot.astype(jnp.float32), xf,
                       preferred_element_type=jnp.float32)
        rel_j = nxyz - xb
        idx_cols.append(sel)
        rel_cols.append(rel_j)
        p1 = jnp.sum(rel_j, axis=0, keepdims=True)
        rx = rel_j[:, 0:1]
        ry = rel_j[:, 1:2]
        rz = rel_j[:, 2:3]
        p2 = jnp.concatenate([
            jnp.sum(rx * rx, axis=0, keepdims=True),
            jnp.sum(rx * ry, axis=0, keepdims=True),
            jnp.sum(rx * rz, axis=0, keepdims=True),
            jnp.sum(ry * ry, axis=0, keepdims=True),
            jnp.sum(ry * rz, axis=0, keepdims=True),
            jnp.sum(rz * rz, axis=0, keepdims=True),
        ], axis=1)
        pj = jnp.concatenate([p1, p2], axis=1)
        s1 = pj if s1 is None else s1 + pj

    idx_ref[...] = jnp.concatenate(idx_cols, axis=1)
    rel_ref[...] = jnp.concatenate(rel_cols, axis=1)

    svec = jnp.pad(s1, ((0, 7), (0, 128 - 9)))
    first = i == 0

    @pl.when(first)
    def _init():
        stats_ref[...] = svec

    @pl.when(jnp.logical_not(first))
    def _acc():
        stats_ref[...] = stats_ref[...] + svec


def _k2_body(g_ref, rel_ref, aq_ref, s0_ref, s1_ref, s2_ref, s3_ref,
             pos_w1, pos_b1, pos_g, pos_beta, pos_w2, pos_b2,
             attn_w1, attn_b1,
             z_ref, vpos_ref, zstats_ref):
    step = pl.program_id(0)
    Minv = 1.0 / float(M_ROWS)

    def st(j):
        return s0_ref[0, j] + s1_ref[0, j] + s2_ref[0, j] + s3_ref[0, j]

    mux = st(0) * Minv
    muy = st(1) * Minv
    muz = st(2) * Minv
    cxx = st(3) * Minv - mux * mux
    cxy = st(4) * Minv - mux * muy
    cxz = st(5) * Minv - mux * muz
    cyy = st(6) * Minv - muy * muy
    cyz = st(7) * Minv - muy * muz
    czz = st(8) * Minv - muz * muz

    w1 = pos_w1[...]
    w1x = w1[0:1, :]
    w1y = w1[1:2, :]
    w1z = w1[2:3, :]
    var_h = (cxx * w1x * w1x + cyy * w1y * w1y + czz * w1z * w1z
             + 2.0 * (cxy * w1x * w1y + cxz * w1x * w1z + cyz * w1y * w1z))
    mean_h = mux * w1x + muy * w1y + muz * w1z + pos_b1[...]
    inv = lax.rsqrt(var_h + 1e-5)
    scale = pos_g[...] * inv
    shift = pos_beta[...] - mean_h * scale

    rel = rel_ref[...]
    rx = rel[:, 0:1]
    ry = rel[:, 1:2]
    rz = rel[:, 2:3]
    h = rx * w1x + ry * w1y + rz * w1z + pos_b1[...]
    h_act = jnp.maximum(h * scale + shift, 0.0)

    Wpz = jnp.dot(pos_w2[...], attn_w1[...], preferred_element_type=jnp.float32)
    pos = jnp.dot(h_act, pos_w2[...], preferred_element_type=jnp.float32)
    posz = jnp.dot(h_act, Wpz, preferred_element_type=jnp.float32)

    g = g_ref[...]
    ak = g[:, 0:C]
    vv = g[:, C:2 * C]

    aq = aq_ref[...]
    aq_rows = jnp.broadcast_to(aq.reshape(MB, 1, C), (MB, K, C)).reshape(MB * K, C)

    bz = attn_b1[...] + jnp.dot(pos_b2[...], attn_w1[...],
                                preferred_element_type=jnp.float32)
    z = aq_rows - ak + posz + bz
    z_ref[...] = z
    vpos_ref[...] = vv + pos + pos_b2[...]

    zsum = jnp.sum(z, axis=0, keepdims=True)
    zsq = jnp.sum(z * z, axis=0, keepdims=True)

    @pl.when(step == 0)
    def _init():
        zstats_ref[...] = jnp.zeros_like(zstats_ref)

    zstats_ref[0:1, :] += zsum
    zstats_ref[1:2, :] += zsq


def _k3_body(z_ref, vpos_ref, z0_ref, z1_ref, z2_ref, z3_ref,
             attn_g, attn_beta, attn_w2, attn_b2, Wout, out_ref):
    Minv = 1.0 / float(M_ROWS)
    zs = z0_ref[...] + z1_ref[...] + z2_ref[...] + z3_ref[...]
    mean_z = zs[0:1, :] * Minv
    var_z = zs[1:2, :] * Minv - mean_z * mean_z
    inv = lax.rsqrt(var_z + 1e-5)
    scale = attn_g[...] * inv
    shift = attn_beta[...] - mean_z * scale

    z_act = jnp.maximum(z_ref[...] * scale + shift, 0.0)
    a = jnp.dot(z_act, attn_w2[...], preferred_element_type=jnp.float32) \
        + attn_b2[...]
    a3 = a.reshape(MB, K, C)
    mx = jnp.max(a3, axis=1, keepdims=True)
    e = jnp.exp(a3 - mx)
    s = jnp.sum(e, axis=1, keepdims=True)
    attn = e / s
    vp = vpos_ref[...].reshape(MB, K, C)
    o = jnp.sum(attn * vp, axis=1)
    out_ref[...] = jnp.dot(o, Wout[...], preferred_element_type=jnp.float32)


def _sc_gather(table, idxflat, nrows):
    from jax.experimental.pallas import tpu_sc as plsc
    info = plsc.get_sparse_core_info()
    NC, NS = info.num_cores, info.num_subcores
    NW = NC * NS
    rows_per_w = nrows // NW
    CHUNK = 128
    n_chunks = rows_per_w // CHUNK
    n_pairs = n_chunks // 2
    mesh = plsc.VectorSubcoreMesh(core_axis_name="c", subcore_axis_name="s")

    @functools.partial(
        pl.kernel, mesh=mesh,
        out_type=jax.ShapeDtypeStruct((nrows, 2 * C), jnp.float32),
        scratch_types=[
            pltpu.VMEM((CHUNK,), jnp.int32),
            pltpu.VMEM((2, CHUNK, 2 * C), jnp.float32),
            pltpu.SemaphoreType.DMA,
            pltpu.SemaphoreType.DMA,
            pltpu.SemaphoreType.DMA,
        ],
    )
    def gk(table_hbm, idx_hbm, out_hbm, idx_v, rows_v, sem_g, sem_w0, sem_w1):
        wid = lax.axis_index("s") * NC + lax.axis_index("c")
        base = wid * rows_per_w

        def body(t, carry):
            off0 = pl.multiple_of(base + (2 * t) * CHUNK, CHUNK)
            off1 = pl.multiple_of(base + (2 * t + 1) * CHUNK, CHUNK)

            @pl.when(t >= 1)
            def _drain():
                pltpu.make_async_copy(
                    rows_v.at[0], out_hbm.at[pl.ds(off0, CHUNK)], sem_w0).wait()
                pltpu.make_async_copy(
                    rows_v.at[1], out_hbm.at[pl.ds(off1, CHUNK)], sem_w1).wait()

            pltpu.sync_copy(idx_hbm.at[pl.ds(off0, CHUNK)], idx_v)
            pltpu.async_copy(table_hbm.at[idx_v], rows_v.at[0], sem_g).wait()
            pltpu.async_copy(rows_v.at[0], out_hbm.at[pl.ds(off0, CHUNK)], sem_w0)

            pltpu.sync_copy(idx_hbm.at[pl.ds(off1, CHUNK)], idx_v)
            pltpu.async_copy(table_hbm.at[idx_v], rows_v.at[1], sem_g).wait()
            pltpu.async_copy(rows_v.at[1], out_hbm.at[pl.ds(off1, CHUNK)], sem_w1)
            return carry

        lax.fori_loop(0, n_pairs, body, 0)
        pltpu.make_async_copy(
            rows_v.at[0], out_hbm.at[pl.ds(base, CHUNK)], sem_w0).wait()
        pltpu.make_async_copy(
            rows_v.at[1], out_hbm.at[pl.ds(base, CHUNK)], sem_w1).wait()

    return gk(table, idxflat)


_WCC = pl.BlockSpec((C, C), lambda i: (0, 0))
_W1C = pl.BlockSpec((1, C), lambda i: (0, 0))
_WST = pl.BlockSpec((8, 128), lambda i: (0, 0))
NK = N * K


def kernel(xyz, feat, Wq, Wk, Wv, Wout, pos_w1, pos_b1, pos_g, pos_beta,
           pos_w2, pos_b2, attn_w1, attn_b1, attn_g, attn_beta,
           attn_w2, attn_b2):
    xyzT = xyz.transpose(0, 2, 1)
    pos_b1r = pos_b1.reshape(1, C)
    pos_gr = pos_g.reshape(1, C)
    pos_betar = pos_beta.reshape(1, C)
    pos_b2r = pos_b2.reshape(1, C)
    attn_b1r = attn_b1.reshape(1, C)
    attn_gr = attn_g.reshape(1, C)
    attn_betar = attn_beta.reshape(1, C)
    attn_b2r = attn_b2.reshape(1, C)

    k1 = pl.pallas_call(
        _k1_body,
        grid=(NSTEPS1,),
        in_specs=[
            pl.BlockSpec((1, NB1, 3), lambda i: (0, i, 0)),
            pl.BlockSpec((1, N, 3), lambda i: (0, 0, 0)),
            pl.BlockSpec((1, 3, N), lambda i: (0, 0, 0)),
            pl.BlockSpec((NB1, C), lambda i: (i, 0)),
            pl.BlockSpec((C, C), lambda i: (0, 0)),
            pl.BlockSpec((C, C), lambda i: (0, 0)),
            pl.BlockSpec((C, C), lambda i: (0, 0)),
            pl.BlockSpec((C, C), lambda i: (0, 0)),
        ],
        out_specs=[
            pl.BlockSpec((NB1, C), lambda i: (i, 0)),
            pl.BlockSpec((NB1, 2 * C), lambda i: (i, 0)),
            pl.BlockSpec((NB1, K), lambda i: (i, 0)),
            pl.BlockSpec((NB1, 3 * K), lambda i: (i, 0)),
            pl.BlockSpec((8, 128), lambda i: (0, 0)),
        ],
        out_shape=[
            jax.ShapeDtypeStruct((N, C), jnp.float32),
            jax.ShapeDtypeStruct((N, 2 * C), jnp.float32),
            jax.ShapeDtypeStruct((N, K), jnp.int32),
            jax.ShapeDtypeStruct((N, 3 * K), jnp.float32),
            jax.ShapeDtypeStruct((8, 128), jnp.float32),
        ],
    )

    aqs, tabs, idxs, rels, stats = [], [], [], [], []
    for b in range(B):
        aq, tab, idx, rel48, st = k1(xyz[b:b + 1], xyz[b:b + 1],
                                     xyzT[b:b + 1], feat[b],
                                     Wq, Wk, Wv, attn_w1)
        aqs.append(aq)
        tabs.append(tab)
        idxs.append(idx)
        rels.append(rel48)
        stats.append(st)

    gs = [_sc_gather(tabs[b], idxs[b].reshape(NK), NK) for b in range(B)]

    nsteps2 = N // MB
    k2 = pl.pallas_call(
        _k2_body,
        grid=(nsteps2,),
        in_specs=[
            pl.BlockSpec((MB * K, 2 * C), lambda i: (i, 0)),
            pl.BlockSpec((MB * K, 3), lambda i: (i, 0)),
            pl.BlockSpec((MB, C), lambda i: (i, 0)),
            _WST, _WST, _WST, _WST,
            pl.BlockSpec((3, C), lambda i: (0, 0)),
            _W1C, _W1C, _W1C, _WCC, _W1C, _WCC, _W1C,
        ],
        out_specs=[
            pl.BlockSpec((MB * K, C), lambda i: (i, 0)),
            pl.BlockSpec((MB * K, C), lambda i: (i, 0)),
            pl.BlockSpec((8, 128), lambda i: (0, 0)),
        ],
        out_shape=[
            jax.ShapeDtypeStruct((NK, C), jnp.float32),
            jax.ShapeDtypeStruct((NK, C), jnp.float32),
            jax.ShapeDtypeStruct((8, 128), jnp.float32),
        ],
    )

    zs, vps, zsts = [], [], []
    for b in range(B):
        z, vpos, zst = k2(gs[b], rels[b].reshape(NK, 3), aqs[b],
                          stats[0], stats[1], stats[2], stats[3],
                          pos_w1, pos_b1r, pos_gr, pos_betar,
                          pos_w2, pos_b2r, attn_w1, attn_b1r)
        zs.append(z)
        vps.append(vpos)
        zsts.append(zst)

    k3 = pl.pallas_call(
        _k3_body,
        grid=(nsteps2,),
        in_specs=[
            pl.BlockSpec((MB * K, C), lambda i: (i, 0)),
            pl.BlockSpec((MB * K, C), lambda i: (i, 0)),
            _WST, _WST, _WST, _WST,
            _W1C, _W1C, _WCC, _W1C, _WCC,
        ],
        out_specs=pl.BlockSpec((MB, C), lambda i: (i, 0)),
        out_shape=jax.ShapeDtypeStruct((N, C), jnp.float32),
    )

    outs = [k3(zs[b], vps[b], zsts[0], zsts[1], zsts[2], zsts[3],
               attn_gr, attn_betar, attn_w2, attn_b2r, Wout)
            for b in range(B)]
    return jnp.stack(outs, axis=0)
